# baseline (speedup 1.0000x reference)
import functools

import jax
import jax.numpy as jnp
from jax import lax
from jax.experimental import pallas as pl
from jax.experimental.pallas import tpu as pltpu
from jax.experimental.pallas import tpu_sc as plsc

NUM_POINTS = 100000
TIME_STEP = 0.01
SPRING_TARGET_DISTANCE = 1.0
SPRING_CONSTANT = 1.0

NC, NS = 2, 16
NW = NC * NS
L = 16

P = 100352
PS = P // NS
RPW = P // NW
CHUNK = 2496


def _rsqrt_nr(x):
    i = lax.bitcast_convert_type(x, jnp.int32)
    i = jnp.int32(0x5F3759DF) - lax.shift_right_logical(i, 1)
    y = lax.bitcast_convert_type(i, jnp.float32)
    xh = x * 0.5
    y = y * (1.5 - xh * y * y)
    y = y * (1.5 - xh * y * y)
    return y


def _force_body(px_h, py_h, pz_h, zer_h, a_h, b_h, acc_h,
                ga0, gb0, ga1, gb1, sa0, sb0, sa1, sb1,
                xa0, ya0, za0, xb0, yb0, zb0,
                xa1, ya1, za1, xb1, yb1, zb1,
                fx0, fy0, fz0, gx0, gy0, gz0,
                fx1, fy1, fz1, gx1, gy1, gz1,
                shx, shy, shz, sax, say, saz,
                mgi0, mgi1, msi0, msi1, mg0, mg1, ms0, ms1):
    c = lax.axis_index("c")
    s = lax.axis_index("s")
    wid = s * NC + c

    gidx = ((ga0, gb0), (ga1, gb1))
    sidx = ((sa0, sb0), (sa1, sb1))
    gbuf = ((xa0, ya0, za0, xb0, yb0, zb0), (xa1, ya1, za1, xb1, yb1, zb1))
    fbuf = ((fx0, fy0, fz0, gx0, gy0, gz0), (fx1, fy1, fz1, gx1, gy1, gz1))
    sem_gi = (mgi0, mgi1)
    sem_si = (msi0, msi1)
    sem_g = (mg0, mg1)
    sem_s = (ms0, ms1)

    stg = pl.ds(s * PS, PS)
    pltpu.sync_copy(px_h.at[stg], shx.at[stg])
    pltpu.sync_copy(py_h.at[stg], shy.at[stg])
    pltpu.sync_copy(pz_h.at[stg], shz.at[stg])
    pltpu.sync_copy(zer_h.at[stg], sax.at[stg])
    pltpu.sync_copy(zer_h.at[stg], say.at[stg])
    pltpu.sync_copy(zer_h.at[stg], saz.at[stg])
    plsc.subcore_barrier()

    n_edges = a_h.shape[0]
    epw = n_edges // NW
    base = wid * epw
    n_chunks = epw // CHUNK

    def gidx_start(off, p):
        pltpu.async_copy(a_h.at[pl.ds(off, CHUNK)], gidx[p][0], sem_gi[p])
        pltpu.async_copy(b_h.at[pl.ds(off, CHUNK)], gidx[p][1], sem_gi[p])

    def gidx_wait(p):
        pltpu.make_async_copy(a_h.at[pl.ds(0, CHUNK)], gidx[p][0], sem_gi[p]).wait()
        pltpu.make_async_copy(b_h.at[pl.ds(0, CHUNK)], gidx[p][1], sem_gi[p]).wait()

    def sidx_start(off, p):
        pltpu.async_copy(a_h.at[pl.ds(off, CHUNK)], sidx[p][0], sem_si[p])
        pltpu.async_copy(b_h.at[pl.ds(off, CHUNK)], sidx[p][1], sem_si[p])

    def sidx_wait(p):
        pltpu.make_async_copy(a_h.at[pl.ds(0, CHUNK)], sidx[p][0], sem_si[p]).wait()
        pltpu.make_async_copy(b_h.at[pl.ds(0, CHUNK)], sidx[p][1], sem_si[p]).wait()

    def gath_start(p):
        ai, bi = gidx[p]
        xa, ya, za, xb, yb, zb = gbuf[p]
        pltpu.async_copy(shx.at[ai], xa, sem_g[p])
        pltpu.async_copy(shy.at[ai], ya, sem_g[p])
        pltpu.async_copy(shz.at[ai], za, sem_g[p])
        pltpu.async_copy(shx.at[bi], xb, sem_g[p])
        pltpu.async_copy(shy.at[bi], yb, sem_g[p])
        pltpu.async_copy(shz.at[bi], zb, sem_g[p])

    def gath_wait(p):
        ai, bi = gidx[p]
        xa, ya, za, xb, yb, zb = gbuf[p]
        pltpu.make_async_copy(shx.at[ai], xa, sem_g[p]).wait()
        pltpu.make_async_copy(shy.at[ai], ya, sem_g[p]).wait()
        pltpu.make_async_copy(shz.at[ai], za, sem_g[p]).wait()
        pltpu.make_async_copy(shx.at[bi], xb, sem_g[p]).wait()
        pltpu.make_async_copy(shy.at[bi], yb, sem_g[p]).wait()
        pltpu.make_async_copy(shz.at[bi], zb, sem_g[p]).wait()

    def scat_start(p):
        ai, bi = sidx[p]
        fx, fy, fz, gx, gy, gz = fbuf[p]
        pltpu.async_copy(fx, sax.at[ai], sem_s[p], add=True)
        pltpu.async_copy(fy, say.at[ai], sem_s[p], add=True)
        pltpu.async_copy(fz, saz.at[ai], sem_s[p], add=True)
        pltpu.async_copy(gx, sax.at[bi], sem_s[p], add=True)
        pltpu.async_copy(gy, say.at[bi], sem_s[p], add=True)
        pltpu.async_copy(gz, saz.at[bi], sem_s[p], add=True)

    def scat_wait(p):
        ai, bi = sidx[p]
        fx, fy, fz, gx, gy, gz = fbuf[p]
        pltpu.make_async_copy(fx, sax.at[ai], sem_s[p]).wait()
        pltpu.make_async_copy(fy, say.at[ai], sem_s[p]).wait()
        pltpu.make_async_copy(fz, saz.at[ai], sem_s[p]).wait()
        pltpu.make_async_copy(gx, sax.at[bi], sem_s[p]).wait()
        pltpu.make_async_copy(gy, say.at[bi], sem_s[p]).wait()
        pltpu.make_async_copy(gz, saz.at[bi], sem_s[p]).wait()

    def compute(p):
        xa, ya, za, xb, yb, zb = gbuf[p]
        fx, fy, fz, gx, gy, gz = fbuf[p]

        def grp(g, carry):
            sl = pl.ds(g * L, L)
            dx = xa[sl] - xb[sl]
            dy = ya[sl] - yb[sl]
            dz = za[sl] - zb[sl]
            d2 = dx * dx + dy * dy + dz * dz
            inv_d = _rsqrt_nr(d2)
            sc = SPRING_CONSTANT * (1.0 - SPRING_TARGET_DISTANCE * inv_d)
            vfx = dx * sc
            vfy = dy * sc
            vfz = dz * sc
            fx[sl] = vfx
            fy[sl] = vfy
            fz[sl] = vfz
            gx[sl] = -vfx
            gy[sl] = -vfy
            gz[sl] = -vfz
            return carry

        lax.fori_loop(0, CHUNK // L, grp, 0, unroll=4)

    gidx_start(base, 0)
    gidx_start(base + CHUNK, 1)
    gidx_wait(0)
    gath_start(0)
    gath_wait(0)
    gidx_wait(1)
    gath_start(1)
    gidx_start(base + 2 * CHUNK, 0)
    sidx_start(base, 0)
    compute(0)
    sidx_wait(0)
    scat_start(0)
    gath_wait(1)
    gidx_wait(0)
    gath_start(0)
    gidx_start(base + 3 * CHUNK, 1)
    sidx_start(base + CHUNK, 1)
    compute(1)
    sidx_wait(1)
    scat_start(1)

    def steady(j, carry):
        off0 = base + (2 * j + 2) * CHUNK
        scat_wait(0)
        gath_wait(0)
        gidx_wait(1)
        gath_start(1)
        gidx_start(off0 + 2 * CHUNK, 0)
        sidx_start(off0, 0)
        compute(0)
        sidx_wait(0)
        scat_start(0)
        scat_wait(1)
        gath_wait(1)
        gidx_wait(0)
        gath_start(0)
        gidx_start(off0 + 3 * CHUNK, 1)
        sidx_start(off0 + CHUNK, 1)
        compute(1)
        sidx_wait(1)
        scat_start(1)
        return carry

    lax.fori_loop(0, (n_chunks - 4) // 2, steady, 0)

    off_t = base + (n_chunks - 2) * CHUNK
    scat_wait(0)
    gath_wait(0)
    gidx_wait(1)
    gath_start(1)
    sidx_start(off_t, 0)
    compute(0)
    sidx_wait(0)
    scat_start(0)
    scat_wait(1)
    gath_wait(1)
    sidx_start(off_t + CHUNK, 1)
    compute(1)
    sidx_wait(1)
    scat_start(1)
    scat_wait(0)
    scat_wait(1)
    plsc.subcore_barrier()

    out0 = c * (3 * P) + s * PS
    pltpu.sync_copy(sax.at[stg], acc_h.at[pl.ds(out0, PS)])
    pltpu.sync_copy(say.at[stg], acc_h.at[pl.ds(out0 + P, PS)])
    pltpu.sync_copy(saz.at[stg], acc_h.at[pl.ds(out0 + 2 * P, PS)])


def _integrate_body(px_h, py_h, pz_h, vx_h, vy_h, vz_h, acc_h,
                    npx_h, npy_h, npz_h, nvx_h, nvy_h, nvz_h,
                    pb, vb, a0, a1):
    c = lax.axis_index("c")
    s = lax.axis_index("s")
    wid = s * NC + c
    base = wid * RPW
    sl = pl.ds(base, RPW)
    ins = ((px_h, vx_h, npx_h, nvx_h),
           (py_h, vy_h, npy_h, nvy_h),
           (pz_h, vz_h, npz_h, nvz_h))
    for d, (p_h, v_h, np_h, nv_h) in enumerate(ins):
        pltpu.sync_copy(p_h.at[sl], pb)
        pltpu.sync_copy(v_h.at[sl], vb)
        pltpu.sync_copy(acc_h.at[pl.ds(d * P + base, RPW)], a0)
        pltpu.sync_copy(acc_h.at[pl.ds(3 * P + d * P + base, RPW)], a1)

        def grp(g, carry):
            w = pl.ds(g * L, L)
            acc = a0[w] + a1[w]
            nv = vb[w] + TIME_STEP * acc
            vb[w] = nv
            pb[w] = pb[w] + TIME_STEP * nv
            return carry

        lax.fori_loop(0, RPW // L, grp, 0, unroll=4)
        pltpu.sync_copy(vb, nv_h.at[sl])
        pltpu.sync_copy(pb, np_h.at[sl])


_mesh = plsc.VectorSubcoreMesh(core_axis_name="c", subcore_axis_name="s",
                               num_cores=NC, num_subcores=NS)

_force_kernel = pl.kernel(
    _force_body,
    out_type=jax.ShapeDtypeStruct((NC * 3 * P,), jnp.float32),
    mesh=_mesh,
    scratch_types=(
        [pltpu.VMEM((CHUNK,), jnp.int32)] * 8
        + [pltpu.VMEM((CHUNK,), jnp.float32)] * 24
        + [pltpu.VMEM_SHARED((P,), jnp.float32)] * 6
        + [pltpu.SemaphoreType.DMA] * 8
    ),
)

_integrate_kernel = pl.kernel(
    _integrate_body,
    out_type=[jax.ShapeDtypeStruct((P,), jnp.float32)] * 6,
    mesh=_mesh,
    scratch_types=[pltpu.VMEM((RPW,), jnp.float32)] * 4,
)


@jax.jit
def kernel(point_position, point_velocity, connections):
    n = point_position.shape[0]
    pad = P - n
    px = jnp.pad(point_position[:, 0], (0, pad))
    py = jnp.pad(point_position[:, 1], (0, pad))
    pz = jnp.pad(point_position[:, 2], (0, pad))
    vx = jnp.pad(point_velocity[:, 0], (0, pad))
    vy = jnp.pad(point_velocity[:, 1], (0, pad))
    vz = jnp.pad(point_velocity[:, 2], (0, pad))
    zer = jnp.zeros((P,), jnp.float32)

    e = connections.shape[0]
    step = NW * CHUNK * 2
    ep = ((e + step - 1) // step) * step
    a = jnp.pad(connections[:, 0], (0, ep - e), constant_values=n)
    b = jnp.pad(connections[:, 1], (0, ep - e), constant_values=n)

    acc = _force_kernel(px, py, pz, zer, a, b)
    npx, npy, npz, nvx, nvy, nvz = _integrate_kernel(px, py, pz, vx, vy, vz, acc)
    new_pos = jnp.stack([npx[:n], npy[:n], npz[:n]], axis=1)
    new_vel = jnp.stack([nvx[:n], nvy[:n], nvz[:n]], axis=1)
    return new_pos, new_vel


# speedup vs baseline: 2.1245x; 2.1245x over previous
import functools

import jax
import jax.numpy as jnp
from jax import lax
from jax.experimental import pallas as pl
from jax.experimental.pallas import tpu as pltpu
from jax.experimental.pallas import tpu_sc as plsc

NUM_POINTS = 100000
TIME_STEP = 0.01
SPRING_TARGET_DISTANCE = 1.0
SPRING_CONSTANT = 1.0

NC, NS = 2, 16
NW = NC * NS
L = 16

P = 100352
PS = P // NS
RPW = P // NW
CHUNK = 2000


def _rsqrt_nr(x):
    i = lax.bitcast_convert_type(x, jnp.int32)
    i = jnp.int32(0x5F3759DF) - lax.shift_right_logical(i, 1)
    y = lax.bitcast_convert_type(i, jnp.float32)
    xh = x * 0.5
    y = y * (1.5 - xh * y * y)
    y = y * (1.5 - xh * y * y)
    return y


def _force_body(px_h, py_h, pz_h, zer_h, a_h, b_h, acc_h,
                ga0, gb0, ga1, gb1, sa0, sb0, sa1, sb1,
                xa0, ya0, za0, xb0, yb0, zb0,
                xa1, ya1, za1, xb1, yb1, zb1,
                fx0, fy0, fz0, gx0, gy0, gz0,
                fx1, fy1, fz1, gx1, gy1, gz1,
                shx, shy, shz, sax, say, saz,
                mgi0, mgi1, msi0, msi1, mg0, mg1, ms0, ms1):
    c = lax.axis_index("c")
    s = lax.axis_index("s")
    wid = s * NC + c

    gidx = ((ga0, gb0), (ga1, gb1))
    sidx = ((sa0, sb0), (sa1, sb1))
    gbuf = ((xa0, ya0, za0, xb0, yb0, zb0), (xa1, ya1, za1, xb1, yb1, zb1))
    fbuf = ((fx0, fy0, fz0, gx0, gy0, gz0), (fx1, fy1, fz1, gx1, gy1, gz1))
    sem_gi = (mgi0, mgi1)
    sem_si = (msi0, msi1)
    sem_g = (mg0, mg1)
    sem_s = (ms0, ms1)

    stg = pl.ds(s * PS, PS)
    pltpu.sync_copy(px_h.at[stg], shx.at[stg])
    pltpu.sync_copy(py_h.at[stg], shy.at[stg])
    pltpu.sync_copy(pz_h.at[stg], shz.at[stg])
    pltpu.sync_copy(zer_h.at[stg], sax.at[stg])
    pltpu.sync_copy(zer_h.at[stg], say.at[stg])
    pltpu.sync_copy(zer_h.at[stg], saz.at[stg])
    plsc.subcore_barrier()

    n_edges = a_h.shape[0]
    epw = n_edges // NW
    base = wid * epw
    n_chunks = epw // CHUNK

    def gidx_start(off, p):
        pltpu.async_copy(a_h.at[pl.ds(off, CHUNK)], gidx[p][0], sem_gi[p])
        pltpu.async_copy(b_h.at[pl.ds(off, CHUNK)], gidx[p][1], sem_gi[p])

    def gidx_wait(p):
        pltpu.make_async_copy(a_h.at[pl.ds(0, CHUNK)], gidx[p][0], sem_gi[p]).wait()
        pltpu.make_async_copy(b_h.at[pl.ds(0, CHUNK)], gidx[p][1], sem_gi[p]).wait()

    def sidx_start(off, p):
        pltpu.async_copy(a_h.at[pl.ds(off, CHUNK)], sidx[p][0], sem_si[p])
        pltpu.async_copy(b_h.at[pl.ds(off, CHUNK)], sidx[p][1], sem_si[p])

    def sidx_wait(p):
        pltpu.make_async_copy(a_h.at[pl.ds(0, CHUNK)], sidx[p][0], sem_si[p]).wait()
        pltpu.make_async_copy(b_h.at[pl.ds(0, CHUNK)], sidx[p][1], sem_si[p]).wait()

    def gath_start(p):
        ai, bi = gidx[p]
        xa, ya, za, xb, yb, zb = gbuf[p]
        pltpu.async_copy(shx.at[ai], xa, sem_g[p])
        pltpu.async_copy(shy.at[ai], ya, sem_g[p])
        pltpu.async_copy(shz.at[ai], za, sem_g[p])
        pltpu.async_copy(shx.at[bi], xb, sem_g[p])
        pltpu.async_copy(shy.at[bi], yb, sem_g[p])
        pltpu.async_copy(shz.at[bi], zb, sem_g[p])

    def gath_wait(p):
        ai, bi = gidx[p]
        xa, ya, za, xb, yb, zb = gbuf[p]
        pltpu.make_async_copy(shx.at[ai], xa, sem_g[p]).wait()
        pltpu.make_async_copy(shy.at[ai], ya, sem_g[p]).wait()
        pltpu.make_async_copy(shz.at[ai], za, sem_g[p]).wait()
        pltpu.make_async_copy(shx.at[bi], xb, sem_g[p]).wait()
        pltpu.make_async_copy(shy.at[bi], yb, sem_g[p]).wait()
        pltpu.make_async_copy(shz.at[bi], zb, sem_g[p]).wait()

    def scat_start(p):
        ai, bi = sidx[p]
        fx, fy, fz, gx, gy, gz = fbuf[p]
        pltpu.async_copy(fx, sax.at[ai], sem_s[p], add=True)
        pltpu.async_copy(fy, say.at[ai], sem_s[p], add=True)
        pltpu.async_copy(fz, saz.at[ai], sem_s[p], add=True)
        pltpu.async_copy(gx, sax.at[bi], sem_s[p], add=True)
        pltpu.async_copy(gy, say.at[bi], sem_s[p], add=True)
        pltpu.async_copy(gz, saz.at[bi], sem_s[p], add=True)

    def scat_wait(p):
        ai, bi = sidx[p]
        fx, fy, fz, gx, gy, gz = fbuf[p]
        pltpu.make_async_copy(fx, sax.at[ai], sem_s[p]).wait()
        pltpu.make_async_copy(fy, say.at[ai], sem_s[p]).wait()
        pltpu.make_async_copy(fz, saz.at[ai], sem_s[p]).wait()
        pltpu.make_async_copy(gx, sax.at[bi], sem_s[p]).wait()
        pltpu.make_async_copy(gy, say.at[bi], sem_s[p]).wait()
        pltpu.make_async_copy(gz, saz.at[bi], sem_s[p]).wait()

    def compute(p):
        xa, ya, za, xb, yb, zb = gbuf[p]
        fx, fy, fz, gx, gy, gz = fbuf[p]

        def grp(g, carry):
            sl = pl.ds(g * L, L)
            dx = xa[sl] - xb[sl]
            dy = ya[sl] - yb[sl]
            dz = za[sl] - zb[sl]
            d2 = dx * dx + dy * dy + dz * dz
            inv_d = _rsqrt_nr(d2)
            sc = SPRING_CONSTANT * (1.0 - SPRING_TARGET_DISTANCE * inv_d)
            vfx = dx * sc
            vfy = dy * sc
            vfz = dz * sc
            fx[sl] = vfx
            fy[sl] = vfy
            fz[sl] = vfz
            gx[sl] = -vfx
            gy[sl] = -vfy
            gz[sl] = -vfz
            return carry

        lax.fori_loop(0, CHUNK // L, grp, 0, unroll=4)

    gidx_start(base, 0)
    gidx_start(base + CHUNK, 1)
    gidx_wait(0)
    gath_start(0)
    gath_wait(0)
    gidx_wait(1)
    gath_start(1)
    gidx_start(base + 2 * CHUNK, 0)
    sidx_start(base, 0)
    compute(0)
    sidx_wait(0)
    scat_start(0)
    gath_wait(1)
    gidx_wait(0)
    gath_start(0)
    gidx_start(base + 3 * CHUNK, 1)
    sidx_start(base + CHUNK, 1)
    compute(1)
    sidx_wait(1)
    scat_start(1)

    def steady(j, carry):
        off0 = base + (2 * j + 2) * CHUNK
        scat_wait(0)
        gath_wait(0)
        gidx_wait(1)
        gath_start(1)
        gidx_start(off0 + 2 * CHUNK, 0)
        sidx_start(off0, 0)
        compute(0)
        sidx_wait(0)
        scat_start(0)
        scat_wait(1)
        gath_wait(1)
        gidx_wait(0)
        gath_start(0)
        gidx_start(off0 + 3 * CHUNK, 1)
        sidx_start(off0 + CHUNK, 1)
        compute(1)
        sidx_wait(1)
        scat_start(1)
        return carry

    lax.fori_loop(0, (n_chunks - 4) // 2, steady, 0)

    off_t = base + (n_chunks - 2) * CHUNK
    scat_wait(0)
    gath_wait(0)
    gidx_wait(1)
    gath_start(1)
    sidx_start(off_t, 0)
    compute(0)
    sidx_wait(0)
    scat_start(0)
    scat_wait(1)
    gath_wait(1)
    sidx_start(off_t + CHUNK, 1)
    compute(1)
    sidx_wait(1)
    scat_start(1)
    scat_wait(0)
    scat_wait(1)
    plsc.subcore_barrier()

    out0 = c * (3 * P) + s * PS
    pltpu.sync_copy(sax.at[stg], acc_h.at[pl.ds(out0, PS)])
    pltpu.sync_copy(say.at[stg], acc_h.at[pl.ds(out0 + P, PS)])
    pltpu.sync_copy(saz.at[stg], acc_h.at[pl.ds(out0 + 2 * P, PS)])


def _integrate_body(px_h, py_h, pz_h, vx_h, vy_h, vz_h, acc_h,
                    npx_h, npy_h, npz_h, nvx_h, nvy_h, nvz_h,
                    pb, vb, a0, a1):
    c = lax.axis_index("c")
    s = lax.axis_index("s")
    wid = s * NC + c
    base = wid * RPW
    sl = pl.ds(base, RPW)
    ins = ((px_h, vx_h, npx_h, nvx_h),
           (py_h, vy_h, npy_h, nvy_h),
           (pz_h, vz_h, npz_h, nvz_h))
    for d, (p_h, v_h, np_h, nv_h) in enumerate(ins):
        pltpu.sync_copy(p_h.at[sl], pb)
        pltpu.sync_copy(v_h.at[sl], vb)
        pltpu.sync_copy(acc_h.at[pl.ds(d * P + base, RPW)], a0)
        pltpu.sync_copy(acc_h.at[pl.ds(3 * P + d * P + base, RPW)], a1)

        def grp(g, carry):
            w = pl.ds(g * L, L)
            acc = a0[w] + a1[w]
            nv = vb[w] + TIME_STEP * acc
            vb[w] = nv
            pb[w] = pb[w] + TIME_STEP * nv
            return carry

        lax.fori_loop(0, RPW // L, grp, 0, unroll=4)
        pltpu.sync_copy(vb, nv_h.at[sl])
        pltpu.sync_copy(pb, np_h.at[sl])


_mesh = plsc.VectorSubcoreMesh(core_axis_name="c", subcore_axis_name="s",
                               num_cores=NC, num_subcores=NS)

_force_kernel = pl.kernel(
    _force_body,
    out_type=jax.ShapeDtypeStruct((NC * 3 * P,), jnp.float32),
    mesh=_mesh,
    scratch_types=(
        [pltpu.VMEM((CHUNK,), jnp.int32)] * 8
        + [pltpu.VMEM((CHUNK,), jnp.float32)] * 24
        + [pltpu.VMEM_SHARED((P,), jnp.float32)] * 6
        + [pltpu.SemaphoreType.DMA] * 8
    ),
)

_integrate_kernel = pl.kernel(
    _integrate_body,
    out_type=[jax.ShapeDtypeStruct((P,), jnp.float32)] * 6,
    mesh=_mesh,
    scratch_types=[pltpu.VMEM((RPW,), jnp.float32)] * 4,
)


@jax.jit
def kernel(point_position, point_velocity, connections):
    n = point_position.shape[0]
    pad = P - n
    px = jnp.pad(point_position[:, 0], (0, pad))
    py = jnp.pad(point_position[:, 1], (0, pad))
    pz = jnp.pad(point_position[:, 2], (0, pad))
    vx = jnp.pad(point_velocity[:, 0], (0, pad))
    vy = jnp.pad(point_velocity[:, 1], (0, pad))
    vz = jnp.pad(point_velocity[:, 2], (0, pad))
    zer = jnp.zeros((P,), jnp.float32)

    e = connections.shape[0]
    step = NW * CHUNK * 2
    ep = ((e + step - 1) // step) * step
    pad_idx = (n + jnp.arange(ep - e, dtype=jnp.int32) % (P - n)).astype(jnp.int32)
    a = jnp.concatenate([connections[:, 0], pad_idx])
    b = jnp.concatenate([connections[:, 1], pad_idx])

    acc = _force_kernel(px, py, pz, zer, a, b)
    npx, npy, npz, nvx, nvy, nvz = _integrate_kernel(px, py, pz, vx, vy, vz, acc)
    new_pos = jnp.stack([npx[:n], npy[:n], npz[:n]], axis=1)
    new_vel = jnp.stack([nvx[:n], nvy[:n], nvz[:n]], axis=1)
    return new_pos, new_vel
